# baseline (speedup 1.0000x reference)
import functools

import jax
import jax.numpy as jnp
from jax import lax
from jax.experimental import pallas as pl
from jax.experimental.pallas import tpu as pltpu
from jax.experimental.pallas import tpu_sc as plsc

N = 10000
D = 128
E = 320000
NP = 10240
NC = 2
NS = 16
NW = NC * NS
EPW = E // NW
CH = 128
NCHF = EPW // CH
TAIL = EPW - NCHF * CH
RPT = NP // NS

_mesh = plsc.VectorSubcoreMesh(core_axis_name="c", subcore_axis_name="s")


def _copy_chunk_idx(dst2, k, src_all, i):
    for j in range(CH // 16):
        dst2[k, pl.ds(j * 16, 16)] = src_all[pl.ds(i * CH + j * 16, 16)]


@functools.partial(
    pl.kernel,
    out_type=jax.ShapeDtypeStruct((NC, NP), jnp.float32),
    mesh=_mesh,
    scratch_types=[
        pltpu.VMEM((EPW,), jnp.int32),
        pltpu.VMEM((2, CH), jnp.int32),
        pltpu.VMEM((TAIL,), jnp.int32),
        pltpu.VMEM((CH,), jnp.float32),
        pltpu.VMEM((RPT,), jnp.float32),
        pltpu.VMEM_SHARED((NP,), jnp.float32),
        pltpu.SemaphoreType.DMA,
        pltpu.SemaphoreType.DMA,
    ],
)
def _deg_kernel(dst_hbm, out_hbm, didx_all, didx2, tidx, ones, zbuf, acc,
                s0, s1):
    cid = lax.axis_index("c")
    sid = lax.axis_index("s")
    wid = cid * NS + sid
    sems = (s0, s1)

    for k in range(CH // 16):
        ones[pl.ds(k * 16, 16)] = jnp.ones((16,), jnp.float32)

    def _zrow(i, _):
        zbuf[pl.ds(i * 16, 16)] = jnp.zeros((16,), jnp.float32)
        return 0

    lax.fori_loop(0, RPT // 16, _zrow, 0)
    pltpu.sync_copy(zbuf, acc.at[pl.ds(sid * RPT, RPT)])
    pltpu.sync_copy(dst_hbm.at[pl.ds(wid * EPW, EPW)], didx_all)
    plsc.subcore_barrier()

    pltpu.async_copy(dst_hbm.at[pl.ds(0, CH)], didx2.at[1], s1)

    def _block(blk, _):
        for u in range(2):
            i = blk * 2 + u
            pltpu.make_async_copy(ones, acc.at[didx2.at[1 - u]],
                                  sems[1 - u]).wait()
            _copy_chunk_idx(didx2, u, didx_all, i)
            pltpu.async_copy(ones, acc.at[didx2.at[u]], sems[u], add=True)
        return 0

    lax.fori_loop(0, NCHF // 2, _block, 0)
    pltpu.make_async_copy(ones, acc.at[didx2.at[1]], s1).wait()

    tidx[pl.ds(0, 16)] = didx_all[pl.ds(NCHF * CH, 16)]
    pltpu.sync_copy(ones.at[pl.ds(0, TAIL)], acc.at[tidx], add=True)

    plsc.subcore_barrier()
    pltpu.sync_copy(acc.at[pl.ds(sid * RPT, RPT)],
                    out_hbm.at[cid, pl.ds(sid * RPT, RPT)])


@functools.partial(
    pl.kernel,
    out_type=jax.ShapeDtypeStruct((NC, NP, D), jnp.float32),
    mesh=_mesh,
    scratch_types=[
        pltpu.VMEM((EPW,), jnp.int32),
        pltpu.VMEM((2, CH), jnp.int32),
        pltpu.VMEM((TAIL,), jnp.int32),
        pltpu.VMEM((TAIL,), jnp.int32),
        pltpu.VMEM((2, CH, D), jnp.float32),
        pltpu.VMEM_SHARED((NP, D), jnp.float32),
        pltpu.SemaphoreType.DMA,
        pltpu.SemaphoreType.DMA,
        pltpu.SemaphoreType.DMA,
        pltpu.SemaphoreType.DMA,
    ],
)
def _prop_kernel(hs_hbm, src_hbm, dst_hbm, zeros_hbm, out_hbm,
                 sidx_all, didx2, tsidx, tdidx, rows, acc,
                 g0, g1, s0, s1):
    cid = lax.axis_index("c")
    sid = lax.axis_index("s")
    wid = cid * NS + sid
    gsem = (g0, g1)
    ssem = (s0, s1)
    ebase = wid * EPW

    r0 = sid * RPT
    pltpu.sync_copy(zeros_hbm.at[pl.ds(r0, RPT)], acc.at[pl.ds(r0, RPT)])
    pltpu.sync_copy(src_hbm.at[pl.ds(ebase, EPW)], sidx_all)
    plsc.subcore_barrier()

    pltpu.async_copy(zeros_hbm.at[pl.ds(0, CH)], rows.at[1], s1)
    pltpu.async_copy(hs_hbm.at[sidx_all.at[pl.ds(0, CH)]], rows.at[0], g0)
    pltpu.async_copy(dst_hbm.at[pl.ds(ebase, CH)], didx2.at[0], g0)

    def _block(blk, _):
        for u in range(2):
            i = blk * 2 + u
            pltpu.make_async_copy(rows.at[1 - u], acc.at[didx2.at[1 - u]],
                                  ssem[1 - u]).wait()

            @pl.when(i + 1 < NCHF)
            def _():
                pltpu.async_copy(
                    hs_hbm.at[sidx_all.at[pl.ds((i + 1) * CH, CH)]],
                    rows.at[1 - u], gsem[1 - u])
                pltpu.async_copy(dst_hbm.at[pl.ds(ebase + (i + 1) * CH, CH)],
                                 didx2.at[1 - u], gsem[1 - u])

            pltpu.make_async_copy(
                hs_hbm.at[sidx_all.at[pl.ds(i * CH, CH)]],
                rows.at[u], gsem[u]).wait()
            pltpu.make_async_copy(dst_hbm.at[pl.ds(ebase + i * CH, CH)],
                                  didx2.at[u], gsem[u]).wait()
            pltpu.async_copy(rows.at[u], acc.at[didx2.at[u]], ssem[u],
                             add=True)
        return 0

    lax.fori_loop(0, NCHF // 2, _block, 0)
    pltpu.make_async_copy(rows.at[1], acc.at[didx2.at[1]], s1).wait()

    tsidx[pl.ds(0, 16)] = sidx_all[pl.ds(NCHF * CH, 16)]
    pltpu.sync_copy(dst_hbm.at[pl.ds(ebase + NCHF * CH, TAIL)], tdidx)
    pltpu.sync_copy(hs_hbm.at[tsidx], rows.at[0].at[pl.ds(0, TAIL)])
    pltpu.sync_copy(rows.at[0].at[pl.ds(0, TAIL)], acc.at[tdidx], add=True)

    plsc.subcore_barrier()
    pltpu.sync_copy(acc.at[pl.ds(r0, RPT)], out_hbm.at[cid, pl.ds(r0, RPT)])


def _mm1_body(degp_ref, x_ref, w1_ref, hs_ref):
    deg = degp_ref[0, :N] + degp_ref[1, :N] + 1.0
    dinv = lax.rsqrt(deg).reshape(N, 1)
    xw = jnp.dot(x_ref[...], w1_ref[...], preferred_element_type=jnp.float32)
    hs_ref[...] = xw * dinv


def _mid_body(p_ref, hs1_ref, degp_ref, b1_ref, g_ref, bt_ref, w2_ref,
              hs2_ref):
    deg = degp_ref[0, :N] + degp_ref[1, :N] + 1.0
    dinv = lax.rsqrt(deg).reshape(N, 1)
    acc = p_ref[0, :N, :] + p_ref[1, :N, :] + hs1_ref[...]
    h = acc * dinv + b1_ref[...].reshape(1, D)
    scale = g_ref[...].reshape(1, D) * (1.0 / jnp.sqrt(1.0 + 1e-5))
    h = h * scale + bt_ref[...].reshape(1, D)
    h = jnp.maximum(h, 0.0)
    hw = jnp.dot(h, w2_ref[...], preferred_element_type=jnp.float32)
    hs2_ref[...] = hw * dinv


def _final_body(p_ref, hs2_ref, degp_ref, b2_ref, l2w_ref, l2b_ref,
                l3w_ref, l3b_ref, out_ref):
    deg = degp_ref[0, :N] + degp_ref[1, :N] + 1.0
    dinv = lax.rsqrt(deg).reshape(N, 1)
    acc = p_ref[0, :N, :] + p_ref[1, :N, :] + hs2_ref[...]
    h2 = acc * dinv + b2_ref[...].reshape(1, D)
    h = jnp.dot(h2, l2w_ref[...], preferred_element_type=jnp.float32)
    h = h + l2b_ref[...].reshape(1, D)
    m = jnp.max(h, axis=1, keepdims=True)
    e = jnp.exp(h - m)
    p = e / jnp.sum(e, axis=1, keepdims=True)
    ent = -jnp.sum(p * jnp.log(p + 1e-9), axis=1, keepdims=True)
    w = 1.0 / (ent + 1e-10)
    wmin = jnp.min(w, axis=0, keepdims=True)
    wmax = jnp.max(w, axis=0, keepdims=True)
    w = (w - wmin) / (wmax - wmin)
    nm = jnp.max(w, axis=0, keepdims=True)
    ew = jnp.exp(w - nm)
    w = ew / jnp.sum(ew, axis=0, keepdims=True)
    pooled = jnp.sum(h * w, axis=0, keepdims=True)
    out_ref[...] = (
        jnp.dot(pooled, l3w_ref[...], preferred_element_type=jnp.float32)
        + l3b_ref[...].reshape(1, D)
    )


def _tc_call(body, out_shape, n_in):
    return pl.pallas_call(
        body,
        out_shape=out_shape,
        in_specs=[pl.BlockSpec(memory_space=pltpu.VMEM)
                  for _ in range(n_in)],
        out_specs=pl.BlockSpec(memory_space=pltpu.VMEM),
    )


@jax.jit
def kernel(x, edge_index, W1, b1, W2, b2, bn_gamma, bn_beta,
           lin2_W, lin2_b, lin3_W, lin3_b):
    src = edge_index[0].astype(jnp.int32)
    dst = edge_index[1].astype(jnp.int32)
    zeros = jnp.zeros((NP, D), jnp.float32)

    degp = _deg_kernel(dst)

    hs1 = _tc_call(_mm1_body, jax.ShapeDtypeStruct((N, D), jnp.float32), 3)(
        degp, x, W1)

    p1 = _prop_kernel(hs1, src, dst, zeros)

    hs2 = _tc_call(_mid_body, jax.ShapeDtypeStruct((N, D), jnp.float32), 7)(
        p1, hs1, degp, b1, bn_gamma, bn_beta, W2)

    p2 = _prop_kernel(hs2, src, dst, zeros)

    graph = _tc_call(_final_body, jax.ShapeDtypeStruct((1, D), jnp.float32), 8)(
        p2, hs2, degp, b2, lin2_W, lin2_b, lin3_W, lin3_b)
    return graph


# speedup vs baseline: 1.0087x; 1.0087x over previous
import functools

import jax
import jax.numpy as jnp
from jax import lax
from jax.experimental import pallas as pl
from jax.experimental.pallas import tpu as pltpu
from jax.experimental.pallas import tpu_sc as plsc

N = 10000
D = 128
E = 320000
NP = 10240
NC = 2
NS = 16
NW = NC * NS
EPW = E // NW
CH = 128
NCHF = EPW // CH
TAIL = EPW - NCHF * CH
RPT = NP // NS

_mesh = plsc.VectorSubcoreMesh(core_axis_name="c", subcore_axis_name="s")


def _copy_chunk_idx(dst2, k, src_all, i):
    for j in range(CH // 16):
        dst2[k, pl.ds(j * 16, 16)] = src_all[pl.ds(i * CH + j * 16, 16)]


@functools.partial(
    pl.kernel,
    out_type=jax.ShapeDtypeStruct((NC, NP), jnp.float32),
    mesh=_mesh,
    scratch_types=[
        pltpu.VMEM((EPW,), jnp.int32),
        pltpu.VMEM((2, CH), jnp.int32),
        pltpu.VMEM((TAIL,), jnp.int32),
        pltpu.VMEM((CH,), jnp.float32),
        pltpu.VMEM((RPT,), jnp.float32),
        pltpu.VMEM_SHARED((NP,), jnp.float32),
        pltpu.SemaphoreType.DMA,
        pltpu.SemaphoreType.DMA,
    ],
)
def _deg_kernel(dst_hbm, out_hbm, didx_all, didx2, tidx, ones, zbuf, acc,
                s0, s1):
    cid = lax.axis_index("c")
    sid = lax.axis_index("s")
    wid = cid * NS + sid
    sems = (s0, s1)

    for k in range(CH // 16):
        ones[pl.ds(k * 16, 16)] = jnp.ones((16,), jnp.float32)

    def _zrow(i, _):
        zbuf[pl.ds(i * 16, 16)] = jnp.zeros((16,), jnp.float32)
        return 0

    lax.fori_loop(0, RPT // 16, _zrow, 0)
    pltpu.sync_copy(zbuf, acc.at[pl.ds(sid * RPT, RPT)])
    pltpu.sync_copy(dst_hbm.at[pl.ds(wid * EPW, EPW)], didx_all)
    plsc.subcore_barrier()

    pltpu.async_copy(dst_hbm.at[pl.ds(0, CH)], didx2.at[1], s1)

    def _block(blk, _):
        for u in range(2):
            i = blk * 2 + u
            pltpu.make_async_copy(ones, acc.at[didx2.at[1 - u]],
                                  sems[1 - u]).wait()
            _copy_chunk_idx(didx2, u, didx_all, i)
            pltpu.async_copy(ones, acc.at[didx2.at[u]], sems[u], add=True)
        return 0

    lax.fori_loop(0, NCHF // 2, _block, 0)
    pltpu.make_async_copy(ones, acc.at[didx2.at[1]], s1).wait()

    tidx[pl.ds(0, 16)] = didx_all[pl.ds(NCHF * CH, 16)]
    pltpu.sync_copy(ones.at[pl.ds(0, TAIL)], acc.at[tidx], add=True)

    plsc.subcore_barrier()
    pltpu.sync_copy(acc.at[pl.ds(sid * RPT, RPT)],
                    out_hbm.at[cid, pl.ds(sid * RPT, RPT)])


@functools.partial(
    pl.kernel,
    out_type=jax.ShapeDtypeStruct((NC, NP, D), jnp.float32),
    mesh=_mesh,
    scratch_types=[
        pltpu.VMEM((EPW,), jnp.int32),
        pltpu.VMEM((2, CH), jnp.int32),
        pltpu.VMEM((TAIL,), jnp.int32),
        pltpu.VMEM((TAIL,), jnp.int32),
        pltpu.VMEM((2, CH, D), jnp.float32),
        pltpu.VMEM_SHARED((NP, D), jnp.float32),
        pltpu.SemaphoreType.DMA,
        pltpu.SemaphoreType.DMA,
        pltpu.SemaphoreType.DMA,
        pltpu.SemaphoreType.DMA,
    ],
)
def _prop_kernel(hs_hbm, src_hbm, dst_hbm, zeros_hbm, out_hbm,
                 sidx_all, didx2, tsidx, tdidx, rows, acc,
                 g0, g1, s0, s1):
    cid = lax.axis_index("c")
    sid = lax.axis_index("s")
    wid = cid * NS + sid
    gsem = (g0, g1)
    ssem = (s0, s1)
    ebase = wid * EPW

    r0 = sid * RPT

    @pl.when(cid == 0)
    def _():
        pltpu.sync_copy(hs_hbm.at[pl.ds(r0, RPT)], acc.at[pl.ds(r0, RPT)])

    @pl.when(cid != 0)
    def _():
        pltpu.sync_copy(zeros_hbm.at[pl.ds(r0, RPT)], acc.at[pl.ds(r0, RPT)])

    pltpu.sync_copy(src_hbm.at[pl.ds(ebase, EPW)], sidx_all)
    plsc.subcore_barrier()

    pltpu.async_copy(zeros_hbm.at[pl.ds(0, CH)], rows.at[1], s1)
    pltpu.async_copy(hs_hbm.at[sidx_all.at[pl.ds(0, CH)]], rows.at[0], g0)
    pltpu.async_copy(dst_hbm.at[pl.ds(ebase, CH)], didx2.at[0], g0)

    def _block(blk, _):
        for u in range(2):
            i = blk * 2 + u
            pltpu.make_async_copy(rows.at[1 - u], acc.at[didx2.at[1 - u]],
                                  ssem[1 - u]).wait()

            @pl.when(i + 1 < NCHF)
            def _():
                pltpu.async_copy(
                    hs_hbm.at[sidx_all.at[pl.ds((i + 1) * CH, CH)]],
                    rows.at[1 - u], gsem[1 - u])
                pltpu.async_copy(dst_hbm.at[pl.ds(ebase + (i + 1) * CH, CH)],
                                 didx2.at[1 - u], gsem[1 - u])

            pltpu.make_async_copy(
                hs_hbm.at[sidx_all.at[pl.ds(i * CH, CH)]],
                rows.at[u], gsem[u]).wait()
            pltpu.make_async_copy(dst_hbm.at[pl.ds(ebase + i * CH, CH)],
                                  didx2.at[u], gsem[u]).wait()
            pltpu.async_copy(rows.at[u], acc.at[didx2.at[u]], ssem[u],
                             add=True)
        return 0

    lax.fori_loop(0, NCHF // 2, _block, 0)
    pltpu.make_async_copy(rows.at[1], acc.at[didx2.at[1]], s1).wait()

    tsidx[pl.ds(0, 16)] = sidx_all[pl.ds(NCHF * CH, 16)]
    pltpu.sync_copy(dst_hbm.at[pl.ds(ebase + NCHF * CH, TAIL)], tdidx)
    pltpu.sync_copy(hs_hbm.at[tsidx], rows.at[0].at[pl.ds(0, TAIL)])
    pltpu.sync_copy(rows.at[0].at[pl.ds(0, TAIL)], acc.at[tdidx], add=True)

    plsc.subcore_barrier()
    pltpu.sync_copy(acc.at[pl.ds(r0, RPT)], out_hbm.at[cid, pl.ds(r0, RPT)])


def _mm1_body(degp_ref, x_ref, w1_ref, hs_ref):
    deg = degp_ref[0, :N] + degp_ref[1, :N] + 1.0
    dinv = lax.rsqrt(deg).reshape(N, 1)
    xw = jnp.dot(x_ref[...], w1_ref[...], preferred_element_type=jnp.float32)
    hs_ref[pl.ds(0, N), :] = xw * dinv
    hs_ref[pl.ds(N, NP - N), :] = jnp.zeros((NP - N, D), jnp.float32)


def _mid_body(p_ref, degp_ref, b1_ref, g_ref, bt_ref, w2_ref, hs2_ref):
    deg = degp_ref[0, :N] + degp_ref[1, :N] + 1.0
    dinv = lax.rsqrt(deg).reshape(N, 1)
    acc = p_ref[0, :N, :] + p_ref[1, :N, :]
    h = acc * dinv + b1_ref[...].reshape(1, D)
    scale = g_ref[...].reshape(1, D) * (1.0 / jnp.sqrt(1.0 + 1e-5))
    h = h * scale + bt_ref[...].reshape(1, D)
    h = jnp.maximum(h, 0.0)
    hw = jnp.dot(h, w2_ref[...], preferred_element_type=jnp.float32)
    hs2_ref[pl.ds(0, N), :] = hw * dinv
    hs2_ref[pl.ds(N, NP - N), :] = jnp.zeros((NP - N, D), jnp.float32)


def _final_body(p_ref, degp_ref, b2_ref, l2w_ref, l2b_ref,
                l3w_ref, l3b_ref, out_ref):
    deg = degp_ref[0, :N] + degp_ref[1, :N] + 1.0
    dinv = lax.rsqrt(deg).reshape(N, 1)
    acc = p_ref[0, :N, :] + p_ref[1, :N, :]
    h2 = acc * dinv + b2_ref[...].reshape(1, D)
    h = jnp.dot(h2, l2w_ref[...], preferred_element_type=jnp.float32)
    h = h + l2b_ref[...].reshape(1, D)
    m = jnp.max(h, axis=1, keepdims=True)
    e = jnp.exp(h - m)
    p = e / jnp.sum(e, axis=1, keepdims=True)
    ent = -jnp.sum(p * jnp.log(p + 1e-9), axis=1, keepdims=True)
    w = 1.0 / (ent + 1e-10)
    wmin = jnp.min(w, axis=0, keepdims=True)
    wmax = jnp.max(w, axis=0, keepdims=True)
    w = (w - wmin) / (wmax - wmin)
    nm = jnp.max(w, axis=0, keepdims=True)
    ew = jnp.exp(w - nm)
    w = ew / jnp.sum(ew, axis=0, keepdims=True)
    pooled = jnp.sum(h * w, axis=0, keepdims=True)
    out_ref[...] = (
        jnp.dot(pooled, l3w_ref[...], preferred_element_type=jnp.float32)
        + l3b_ref[...].reshape(1, D)
    )


def _tc_call(body, out_shape, n_in):
    return pl.pallas_call(
        body,
        out_shape=out_shape,
        in_specs=[pl.BlockSpec(memory_space=pltpu.VMEM)
                  for _ in range(n_in)],
        out_specs=pl.BlockSpec(memory_space=pltpu.VMEM),
    )


@jax.jit
def kernel(x, edge_index, W1, b1, W2, b2, bn_gamma, bn_beta,
           lin2_W, lin2_b, lin3_W, lin3_b):
    src = edge_index[0].astype(jnp.int32)
    dst = edge_index[1].astype(jnp.int32)
    zeros = jnp.zeros((NP, D), jnp.float32)

    degp = _deg_kernel(dst)

    hs1 = _tc_call(_mm1_body, jax.ShapeDtypeStruct((NP, D), jnp.float32), 3)(
        degp, x, W1)

    p1 = _prop_kernel(hs1, src, dst, zeros)

    hs2 = _tc_call(_mid_body, jax.ShapeDtypeStruct((NP, D), jnp.float32), 6)(
        p1, degp, b1, bn_gamma, bn_beta, W2)

    p2 = _prop_kernel(hs2, src, dst, zeros)

    graph = _tc_call(_final_body, jax.ShapeDtypeStruct((1, D), jnp.float32), 7)(
        p2, degp, b2, lin2_W, lin2_b, lin3_W, lin3_b)
    return graph


# speedup vs baseline: 1.0098x; 1.0011x over previous
import functools

import jax
import jax.numpy as jnp
from jax import lax
from jax.experimental import pallas as pl
from jax.experimental.pallas import tpu as pltpu
from jax.experimental.pallas import tpu_sc as plsc

N = 10000
D = 128
E = 320000
NP = 10240
NC = 2
NS = 16
NW = NC * NS
EPW = E // NW
CH = 128
NCHF = EPW // CH
TAIL = EPW - NCHF * CH
RPT = NP // NS

_mesh = plsc.VectorSubcoreMesh(core_axis_name="c", subcore_axis_name="s")


def _copy_chunk_idx(dst2, k, src_all, i):
    for j in range(CH // 16):
        dst2[k, pl.ds(j * 16, 16)] = src_all[pl.ds(i * CH + j * 16, 16)]


@functools.partial(
    pl.kernel,
    out_type=jax.ShapeDtypeStruct((NC, NP), jnp.float32),
    mesh=_mesh,
    scratch_types=[
        pltpu.VMEM((EPW,), jnp.int32),
        pltpu.VMEM((2, CH), jnp.int32),
        pltpu.VMEM((TAIL,), jnp.int32),
        pltpu.VMEM((CH,), jnp.float32),
        pltpu.VMEM((RPT,), jnp.float32),
        pltpu.VMEM_SHARED((NP,), jnp.float32),
        pltpu.SemaphoreType.DMA,
        pltpu.SemaphoreType.DMA,
    ],
)
def _deg_kernel(dst_hbm, out_hbm, didx_all, didx2, tidx, ones, zbuf, acc,
                s0, s1):
    cid = lax.axis_index("c")
    sid = lax.axis_index("s")
    wid = cid * NS + sid
    sems = (s0, s1)

    for k in range(CH // 16):
        ones[pl.ds(k * 16, 16)] = jnp.ones((16,), jnp.float32)

    def _zrow(i, _):
        zbuf[pl.ds(i * 16, 16)] = jnp.zeros((16,), jnp.float32)
        return 0

    lax.fori_loop(0, RPT // 16, _zrow, 0)
    pltpu.sync_copy(zbuf, acc.at[pl.ds(sid * RPT, RPT)])
    pltpu.sync_copy(dst_hbm.at[pl.ds(wid * EPW, EPW)], didx_all)
    plsc.subcore_barrier()

    pltpu.async_copy(dst_hbm.at[pl.ds(0, CH)], didx2.at[1], s1)

    def _block(blk, _):
        for u in range(2):
            i = blk * 2 + u
            pltpu.make_async_copy(ones, acc.at[didx2.at[1 - u]],
                                  sems[1 - u]).wait()
            _copy_chunk_idx(didx2, u, didx_all, i)
            pltpu.async_copy(ones, acc.at[didx2.at[u]], sems[u], add=True)
        return 0

    lax.fori_loop(0, NCHF // 2, _block, 0, unroll=3)
    pltpu.make_async_copy(ones, acc.at[didx2.at[1]], s1).wait()

    tidx[pl.ds(0, 16)] = didx_all[pl.ds(NCHF * CH, 16)]
    pltpu.sync_copy(ones.at[pl.ds(0, TAIL)], acc.at[tidx], add=True)

    plsc.subcore_barrier()
    pltpu.sync_copy(acc.at[pl.ds(sid * RPT, RPT)],
                    out_hbm.at[cid, pl.ds(sid * RPT, RPT)])


@functools.partial(
    pl.kernel,
    out_type=jax.ShapeDtypeStruct((NC, NP, D), jnp.float32),
    mesh=_mesh,
    scratch_types=[
        pltpu.VMEM((EPW,), jnp.int32),
        pltpu.VMEM((2, CH), jnp.int32),
        pltpu.VMEM((TAIL,), jnp.int32),
        pltpu.VMEM((TAIL,), jnp.int32),
        pltpu.VMEM((2, CH, D), jnp.float32),
        pltpu.VMEM_SHARED((NP, D), jnp.float32),
        pltpu.SemaphoreType.DMA,
        pltpu.SemaphoreType.DMA,
        pltpu.SemaphoreType.DMA,
        pltpu.SemaphoreType.DMA,
    ],
)
def _prop_kernel(hs_hbm, src_hbm, dst_hbm, zeros_hbm, out_hbm,
                 sidx_all, didx2, tsidx, tdidx, rows, acc,
                 g0, g1, s0, s1):
    cid = lax.axis_index("c")
    sid = lax.axis_index("s")
    wid = cid * NS + sid
    gsem = (g0, g1)
    ssem = (s0, s1)
    ebase = wid * EPW

    r0 = sid * RPT

    @pl.when(cid == 0)
    def _():
        pltpu.sync_copy(hs_hbm.at[pl.ds(r0, RPT)], acc.at[pl.ds(r0, RPT)])

    @pl.when(cid != 0)
    def _():
        pltpu.sync_copy(zeros_hbm.at[pl.ds(r0, RPT)], acc.at[pl.ds(r0, RPT)])

    pltpu.sync_copy(src_hbm.at[pl.ds(ebase, EPW)], sidx_all)
    plsc.subcore_barrier()

    pltpu.async_copy(zeros_hbm.at[pl.ds(0, CH)], rows.at[1], s1)
    pltpu.async_copy(hs_hbm.at[sidx_all.at[pl.ds(0, CH)]], rows.at[0], g0)
    pltpu.async_copy(dst_hbm.at[pl.ds(ebase, CH)], didx2.at[0], g0)

    def _block(blk, _):
        for u in range(2):
            i = blk * 2 + u
            pltpu.make_async_copy(rows.at[1 - u], acc.at[didx2.at[1 - u]],
                                  ssem[1 - u]).wait()

            @pl.when(i + 1 < NCHF)
            def _():
                pltpu.async_copy(
                    hs_hbm.at[sidx_all.at[pl.ds((i + 1) * CH, CH)]],
                    rows.at[1 - u], gsem[1 - u])
                pltpu.async_copy(dst_hbm.at[pl.ds(ebase + (i + 1) * CH, CH)],
                                 didx2.at[1 - u], gsem[1 - u])

            pltpu.make_async_copy(
                hs_hbm.at[sidx_all.at[pl.ds(i * CH, CH)]],
                rows.at[u], gsem[u]).wait()
            pltpu.make_async_copy(dst_hbm.at[pl.ds(ebase + i * CH, CH)],
                                  didx2.at[u], gsem[u]).wait()
            pltpu.async_copy(rows.at[u], acc.at[didx2.at[u]], ssem[u],
                             add=True)
        return 0

    lax.fori_loop(0, NCHF // 2, _block, 0)
    pltpu.make_async_copy(rows.at[1], acc.at[didx2.at[1]], s1).wait()

    tsidx[pl.ds(0, 16)] = sidx_all[pl.ds(NCHF * CH, 16)]
    pltpu.sync_copy(dst_hbm.at[pl.ds(ebase + NCHF * CH, TAIL)], tdidx)
    pltpu.sync_copy(hs_hbm.at[tsidx], rows.at[0].at[pl.ds(0, TAIL)])
    pltpu.sync_copy(rows.at[0].at[pl.ds(0, TAIL)], acc.at[tdidx], add=True)

    plsc.subcore_barrier()
    pltpu.sync_copy(acc.at[pl.ds(r0, RPT)], out_hbm.at[cid, pl.ds(r0, RPT)])


def _mm1_body(degp_ref, x_ref, w1_ref, hs_ref):
    deg = degp_ref[0, :N] + degp_ref[1, :N] + 1.0
    dinv = lax.rsqrt(deg).reshape(N, 1)
    xw = jnp.dot(x_ref[...], w1_ref[...], preferred_element_type=jnp.float32)
    hs_ref[pl.ds(0, N), :] = xw * dinv
    hs_ref[pl.ds(N, NP - N), :] = jnp.zeros((NP - N, D), jnp.float32)


def _mid_body(p_ref, degp_ref, b1_ref, g_ref, bt_ref, w2_ref, hs2_ref):
    deg = degp_ref[0, :N] + degp_ref[1, :N] + 1.0
    dinv = lax.rsqrt(deg).reshape(N, 1)
    acc = p_ref[0, :N, :] + p_ref[1, :N, :]
    h = acc * dinv + b1_ref[...].reshape(1, D)
    scale = g_ref[...].reshape(1, D) * (1.0 / jnp.sqrt(1.0 + 1e-5))
    h = h * scale + bt_ref[...].reshape(1, D)
    h = jnp.maximum(h, 0.0)
    hw = jnp.dot(h, w2_ref[...], preferred_element_type=jnp.float32)
    hs2_ref[pl.ds(0, N), :] = hw * dinv
    hs2_ref[pl.ds(N, NP - N), :] = jnp.zeros((NP - N, D), jnp.float32)


def _final_body(p_ref, degp_ref, b2_ref, l2w_ref, l2b_ref,
                l3w_ref, l3b_ref, out_ref):
    deg = degp_ref[0, :N] + degp_ref[1, :N] + 1.0
    dinv = lax.rsqrt(deg).reshape(N, 1)
    acc = p_ref[0, :N, :] + p_ref[1, :N, :]
    h2 = acc * dinv + b2_ref[...].reshape(1, D)
    h = jnp.dot(h2, l2w_ref[...], preferred_element_type=jnp.float32)
    h = h + l2b_ref[...].reshape(1, D)
    m = jnp.max(h, axis=1, keepdims=True)
    e = jnp.exp(h - m)
    p = e / jnp.sum(e, axis=1, keepdims=True)
    ent = -jnp.sum(p * jnp.log(p + 1e-9), axis=1, keepdims=True)
    w = 1.0 / (ent + 1e-10)
    wmin = jnp.min(w, axis=0, keepdims=True)
    wmax = jnp.max(w, axis=0, keepdims=True)
    w = (w - wmin) / (wmax - wmin)
    nm = jnp.max(w, axis=0, keepdims=True)
    ew = jnp.exp(w - nm)
    w = ew / jnp.sum(ew, axis=0, keepdims=True)
    pooled = jnp.sum(h * w, axis=0, keepdims=True)
    out_ref[...] = (
        jnp.dot(pooled, l3w_ref[...], preferred_element_type=jnp.float32)
        + l3b_ref[...].reshape(1, D)
    )


def _tc_call(body, out_shape, n_in):
    return pl.pallas_call(
        body,
        out_shape=out_shape,
        in_specs=[pl.BlockSpec(memory_space=pltpu.VMEM)
                  for _ in range(n_in)],
        out_specs=pl.BlockSpec(memory_space=pltpu.VMEM),
    )


@jax.jit
def kernel(x, edge_index, W1, b1, W2, b2, bn_gamma, bn_beta,
           lin2_W, lin2_b, lin3_W, lin3_b):
    src = edge_index[0].astype(jnp.int32)
    dst = edge_index[1].astype(jnp.int32)
    zeros = jnp.zeros((NP, D), jnp.float32)

    degp = _deg_kernel(dst)

    hs1 = _tc_call(_mm1_body, jax.ShapeDtypeStruct((NP, D), jnp.float32), 3)(
        degp, x, W1)

    p1 = _prop_kernel(hs1, src, dst, zeros)

    hs2 = _tc_call(_mid_body, jax.ShapeDtypeStruct((NP, D), jnp.float32), 6)(
        p1, degp, b1, bn_gamma, bn_beta, W2)

    p2 = _prop_kernel(hs2, src, dst, zeros)

    graph = _tc_call(_final_body, jax.ShapeDtypeStruct((1, D), jnp.float32), 7)(
        p2, degp, b2, lin2_W, lin2_b, lin3_W, lin3_b)
    return graph
